# baseline (speedup 1.0000x reference)
import functools

import jax
import jax.numpy as jnp
from jax import lax
from jax.experimental import pallas as pl
from jax.experimental.pallas import tpu as pltpu
from jax.experimental.pallas import tpu_sc as plsc

DIM = 64
NC = 2
NS = 16
NW = NC * NS
BLK = 128
NGBUF = 4
NOBUF = 2


def _make_gather(L, B):
    assert B % (NW * BLK) == 0 and B // NW == BLK
    assert L % NGBUF == 0
    mesh = plsc.VectorSubcoreMesh(core_axis_name="c", subcore_axis_name="s")

    @functools.partial(
        pl.kernel,
        mesh=mesh,
        out_type=jax.ShapeDtypeStruct((L, DIM, B), jnp.float32),
        compiler_params=pltpu.CompilerParams(
            use_tc_tiling_on_sc=False, needs_layout_passes=False
        ),
        scratch_types=[
            pltpu.VMEM((L, BLK), jnp.int32),
            pltpu.VMEM((NGBUF, BLK, DIM), jnp.float32),
            pltpu.VMEM((NOBUF, DIM, BLK), jnp.float32),
            pltpu.SemaphoreType.DMA,
            pltpu.SemaphoreType.DMA,
            pltpu.SemaphoreType.DMA,
            pltpu.SemaphoreType.DMA,
            pltpu.SemaphoreType.DMA,
            pltpu.SemaphoreType.DMA,
        ],
    )
    def gather_kernel(idx_hbm, table_hbm, out_hbm, idx_v, rows_v, stage_v,
                      g0, g1, g2, g3, o0, o1):
        gsem = (g0, g1, g2, g3)
        osem = (o0, o1)
        wid = lax.axis_index("s") * NC + lax.axis_index("c")
        b0 = wid * BLK

        pltpu.sync_copy(idx_hbm.at[:, pl.ds(b0, BLK)], idx_v)

        def fire_gather(l, p):
            return pltpu.async_copy(
                table_hbm.at[idx_v.at[l]], rows_v.at[p], gsem[p]
            )

        for p in range(NGBUF):
            fire_gather(p, p)

        lane = lax.iota(jnp.int32, 16)

        def body(g, carry):
            for l in range(NGBUF):
                labs = g * NGBUF + l
                p = l
                q = l % NOBUF

                pltpu.make_async_copy(
                    table_hbm.at[idx_v.at[labs]], rows_v.at[p], gsem[p]
                ).wait()

                @pl.when(jnp.logical_or(l >= NOBUF, g > 0))
                def _drain_out():
                    pltpu.make_async_copy(
                        stage_v.at[q], out_hbm.at[labs, :, pl.ds(b0, BLK)],
                        osem[q],
                    ).wait()

                def col(d, c2):
                    for ib in range(BLK // 16):
                        rid = lane + (16 * ib)
                        vals = plsc.load_gather(
                            rows_v.at[p], [rid, jnp.full((16,), d, jnp.int32)]
                        )
                        plsc.store_scatter(
                            stage_v.at[q],
                            [jnp.full((16,), d, jnp.int32), rid],
                            vals,
                        )
                    return c2

                lax.fori_loop(0, DIM, col, 0)

                pltpu.async_copy(
                    stage_v.at[q], out_hbm.at[labs, :, pl.ds(b0, BLK)], osem[q]
                )

                @pl.when(labs + NGBUF < L)
                def _refill():
                    fire_gather(labs + NGBUF, p)
            return carry

        lax.fori_loop(0, L // NGBUF, body, 0)
        for q in range(NOBUF):
            pltpu.make_async_copy(
                stage_v.at[q], out_hbm.at[0, :, pl.ds(b0, BLK)], osem[q]
            ).wait()

    return gather_kernel


def kernel(input, weight):
    B, L = input.shape
    idx_t = input.T.astype(jnp.int32)
    out_t = _make_gather(L, B)(idx_t, weight)
    return jnp.transpose(out_t, (2, 0, 1))


# speedup vs baseline: 2.1229x; 2.1229x over previous
import functools

import jax
import jax.numpy as jnp
from jax import lax
from jax.experimental import pallas as pl
from jax.experimental.pallas import tpu as pltpu
from jax.experimental.pallas import tpu_sc as plsc

DIM = 64
NC = 2
NS = 16
NW = NC * NS
BLK = 128
NGBUF = 4
NOBUF = 2


def _make_gather(L, B):
    assert B % (NW * BLK) == 0 and B // NW == BLK
    assert L % NGBUF == 0
    mesh = plsc.VectorSubcoreMesh(core_axis_name="c", subcore_axis_name="s")

    @functools.partial(
        pl.kernel,
        mesh=mesh,
        out_type=jax.ShapeDtypeStruct((L, DIM, B), jnp.float32),
        compiler_params=pltpu.CompilerParams(
            use_tc_tiling_on_sc=False, needs_layout_passes=False
        ),
        scratch_types=[
            pltpu.VMEM((L, BLK), jnp.int32),
            pltpu.VMEM((NGBUF, BLK, DIM), jnp.float32),
            pltpu.VMEM((NOBUF, DIM, BLK + 1), jnp.float32),
            pltpu.SemaphoreType.DMA,
            pltpu.SemaphoreType.DMA,
            pltpu.SemaphoreType.DMA,
            pltpu.SemaphoreType.DMA,
            pltpu.SemaphoreType.DMA,
            pltpu.SemaphoreType.DMA,
        ],
    )
    def gather_kernel(idx_hbm, table_hbm, out_hbm, idx_v, rows_v, stage_v,
                      g0, g1, g2, g3, o0, o1):
        gsem = (g0, g1, g2, g3)
        osem = (o0, o1)
        wid = lax.axis_index("s") * NC + lax.axis_index("c")
        b0 = wid * BLK

        pltpu.sync_copy(idx_hbm.at[:, pl.ds(b0, BLK)], idx_v)

        def fire_gather(l, p):
            return pltpu.async_copy(
                table_hbm.at[idx_v.at[l]], rows_v.at[p], gsem[p]
            )

        for p in range(NGBUF):
            fire_gather(p, p)

        lane = lax.iota(jnp.int32, 16)
        rids = [lane + (16 * ib) for ib in range(BLK // 16)]

        def body(g, carry):
            for l in range(NGBUF):
                labs = g * NGBUF + l
                p = l
                q = l % NOBUF

                pltpu.make_async_copy(
                    table_hbm.at[idx_v.at[labs]], rows_v.at[p], gsem[p]
                ).wait()

                @pl.when(jnp.logical_or(l >= NOBUF, g > 0))
                def _drain_out():
                    pltpu.make_async_copy(
                        stage_v.at[q, :, pl.ds(0, BLK)],
                        out_hbm.at[labs, :, pl.ds(b0, BLK)], osem[q],
                    ).wait()

                @plsc.parallel_loop(0, BLK, unroll=8)
                def _row(i):
                    iv = jnp.full((16,), i, jnp.int32)
                    for j in range(DIM // 16):
                        vec = rows_v[p, i, pl.ds(16 * j, 16)]
                        plsc.store_scatter(stage_v.at[q], [rids[j], iv], vec)

                pltpu.async_copy(
                    stage_v.at[q, :, pl.ds(0, BLK)],
                    out_hbm.at[labs, :, pl.ds(b0, BLK)], osem[q]
                )

                @pl.when(labs + NGBUF < L)
                def _refill():
                    fire_gather(labs + NGBUF, p)
            return carry

        lax.fori_loop(0, L // NGBUF, body, 0)
        for q in range(NOBUF):
            pltpu.make_async_copy(
                stage_v.at[q, :, pl.ds(0, BLK)],
                out_hbm.at[0, :, pl.ds(b0, BLK)], osem[q]
            ).wait()

    return gather_kernel


def kernel(input, weight):
    B, L = input.shape
    idx_t = input.T.astype(jnp.int32)
    out_t = _make_gather(L, B)(idx_t, weight)
    return jnp.transpose(out_t, (2, 0, 1))
